# baseline (speedup 1.0000x reference)
import functools

import jax
import jax.numpy as jnp
from jax import lax
from jax.experimental import pallas as pl
from jax.experimental.pallas import tpu as pltpu
from jax.experimental.pallas import tpu_sc as plsc

NC = 2
NS = 16
NW = NC * NS

T = 200
B = 16384
H = 32
TR = T // 8
JB = B // 128
JPW = JB // NW
HB = H // 8


@jax.jit
def _embed_lookup(idx4, table):
    mesh = plsc.VectorSubcoreMesh(core_axis_name="c", subcore_axis_name="s")

    @functools.partial(
        pl.kernel,
        out_type=jax.ShapeDtypeStruct((T, HB, JB * 1024), jnp.float32),
        mesh=mesh,
        scratch_types=[
            pltpu.VMEM((8, 128), jnp.int32),
            pltpu.VMEM((2, 4, 128, H), jnp.float32),
            pltpu.VMEM((2, 4, HB, 1024), jnp.float32),
            pltpu.SemaphoreType.DMA((2,)),
            pltpu.SemaphoreType.DMA((2,)),
        ],
        compiler_params=pltpu.CompilerParams(use_tc_tiling_on_sc=False, needs_layout_passes=False),
    )
    def body(idx_hbm, table_hbm, out_hbm, idx_v, rows_v, trans_v, gsem, osem):
        wid = lax.axis_index("s") * NC + lax.axis_index("c")
        jb0 = wid * JPW

        iota = lax.iota(jnp.int32, 16)

        def wait_write(hf):
            pltpu.make_async_copy(
                trans_v.at[hf],
                out_hbm.at[pl.ds(0, 4), :, pl.ds(0, 1024)],
                osem.at[hf],
            ).wait()

        def tblock(s, carry):
            jb = jb0 + s // TR
            tr = s % TR
            t0 = tr * 8
            pltpu.sync_copy(idx_hbm.at[tr, jb], idx_v)
            copies = [
                [
                    pltpu.async_copy(
                        table_hbm.at[idx_v.at[hf * 4 + u]],
                        rows_v.at[hf, u],
                        gsem.at[hf],
                    )
                    for u in range(4)
                ]
                for hf in range(2)
            ]
            for hf in range(2):
                for c in copies[hf]:
                    c.wait()
                @pl.when(s >= 1)
                def _():
                    wait_write(hf)

                def _bloop(b0, c2):
                    bvec = iota + b0 * 16
                    for u in range(4):
                        for h in range(H):
                            hv = jnp.full((16,), h, jnp.int32)
                            x = plsc.load_gather(rows_v.at[hf, u], [bvec, hv])
                            trans_v[hf, u, h >> 3,
                                    pl.ds((h & 7) * 128 + b0 * 16, 16)] = x
                    return c2

                lax.fori_loop(0, 8, _bloop, 0)
                pltpu.async_copy(
                    trans_v.at[hf],
                    out_hbm.at[pl.ds(t0 + hf * 4, 4), :, pl.ds(jb * 1024, 1024)],
                    osem.at[hf],
                )
            return carry

        lax.fori_loop(0, JPW * TR, tblock, 0)
        for hf in range(2):
            wait_write(hf)

    return body(idx4, table)


def kernel(input_ids, W_embed):
    idx4 = input_ids.T.reshape(TR, 8, JB, 128).transpose(0, 2, 1, 3)
    out2 = _embed_lookup(idx4, W_embed)
    out = (
        out2.reshape(T, HB, JB, 8, 128)
        .transpose(2, 4, 0, 1, 3)
        .reshape(B, T, H)
    )
    return out


# speedup vs baseline: 3.6867x; 3.6867x over previous
import functools

import jax
import jax.numpy as jnp
from jax import lax
from jax.experimental import pallas as pl
from jax.experimental.pallas import tpu as pltpu
from jax.experimental.pallas import tpu_sc as plsc

NC = 2
NS = 16
NW = NC * NS

T = 200
B = 16384
H = 32
TR = T // 8
JB = B // 128
JPW = JB // NW
HB = H // 8


@jax.jit
def _embed_lookup(idx4, table):
    mesh = plsc.VectorSubcoreMesh(core_axis_name="c", subcore_axis_name="s")

    @functools.partial(
        pl.kernel,
        out_type=jax.ShapeDtypeStruct((T, HB, JB * 1024), jnp.float32),
        mesh=mesh,
        scratch_types=[
            pltpu.VMEM((8, 128), jnp.int32),
            pltpu.VMEM((2, 4, 128, H), jnp.float32),
            pltpu.VMEM((2, 4, HB, 1024), jnp.float32),
            pltpu.SemaphoreType.DMA((2,)),
            pltpu.SemaphoreType.DMA((2,)),
        ],
        compiler_params=pltpu.CompilerParams(use_tc_tiling_on_sc=False, needs_layout_passes=False),
    )
    def body(idx_hbm, table_hbm, out_hbm, idx_v, rows_v, trans_v, gsem, osem):
        wid = lax.axis_index("s") * NC + lax.axis_index("c")
        jb0 = wid * JPW

        iota = lax.iota(jnp.int32, 16)

        def wait_write(hf):
            pltpu.make_async_copy(
                trans_v.at[hf],
                out_hbm.at[pl.ds(0, 4), :, pl.ds(0, 1024)],
                osem.at[hf],
            ).wait()

        def tblock(s, carry):
            jb = jb0 + s // TR
            tr = s % TR
            t0 = tr * 8
            pltpu.sync_copy(idx_hbm.at[tr, jb], idx_v)
            copies = [
                [
                    pltpu.async_copy(
                        table_hbm.at[idx_v.at[hf * 4 + u]],
                        rows_v.at[hf, u],
                        gsem.at[hf],
                    )
                    for u in range(4)
                ]
                for hf in range(2)
            ]
            for hf in range(2):
                for c in copies[hf]:
                    c.wait()
                @pl.when(s >= 1)
                def _():
                    wait_write(hf)

                plsc.subcore_barrier()

                @functools.partial(plsc.parallel_loop, 0, 8, unroll=2)
                def _(b0):
                    bvec = iota + b0 * 16
                    for u in range(4):
                        for h in range(H):
                            hv = jnp.full((16,), h, jnp.int32)
                            x = plsc.load_gather(rows_v.at[hf, u], [bvec, hv])
                            trans_v[hf, u, h >> 3,
                                    pl.ds((h & 7) * 128 + b0 * 16, 16)] = x

                plsc.subcore_barrier()
                pltpu.async_copy(
                    trans_v.at[hf],
                    out_hbm.at[pl.ds(t0 + hf * 4, 4), :, pl.ds(jb * 1024, 1024)],
                    osem.at[hf],
                )
            return carry

        lax.fori_loop(0, JPW * TR, tblock, 0)
        for hf in range(2):
            wait_write(hf)

    return body(idx4, table)


def kernel(input_ids, W_embed):
    idx4 = input_ids.T.reshape(TR, 8, JB, 128).transpose(0, 2, 1, 3)
    out2 = _embed_lookup(idx4, W_embed)
    out = (
        out2.reshape(T, HB, JB, 8, 128)
        .transpose(2, 4, 0, 1, 3)
        .reshape(B, T, H)
    )
    return out


# speedup vs baseline: 4.0667x; 1.1031x over previous
import functools

import jax
import jax.numpy as jnp
from jax import lax
from jax.experimental import pallas as pl
from jax.experimental.pallas import tpu as pltpu
from jax.experimental.pallas import tpu_sc as plsc

NC = 2
NS = 16
NW = NC * NS

T = 200
B = 16384
H = 32
TR = T // 8
JB = B // 128
JPW = JB // NW
HB = H // 8
NTB = JPW * TR


@jax.jit
def _embed_lookup(idx4, table):
    mesh = plsc.VectorSubcoreMesh(core_axis_name="c", subcore_axis_name="s")

    @functools.partial(
        pl.kernel,
        out_type=jax.ShapeDtypeStruct((T, HB, JB * 1024), jnp.float32),
        mesh=mesh,
        scratch_types=[
            pltpu.VMEM((2, 8, 128), jnp.int32),
            pltpu.VMEM((2, 2, 4, 128, H), jnp.float32),
            pltpu.VMEM((2, 4, HB, 1024), jnp.float32),
            pltpu.SemaphoreType.DMA((2,)),
            pltpu.SemaphoreType.DMA((2, 2)),
            pltpu.SemaphoreType.DMA((2,)),
        ],
        compiler_params=pltpu.CompilerParams(
            use_tc_tiling_on_sc=False, needs_layout_passes=False
        ),
    )
    def body(idx_hbm, table_hbm, out_hbm, idx_v, rows_v, trans_v,
             isem, gsem, osem):
        wid = lax.axis_index("s") * NC + lax.axis_index("c")
        jb0 = wid * JPW
        iota = lax.iota(jnp.int32, 16)

        def idx_start(s):
            pltpu.async_copy(
                idx_hbm.at[s % TR, jb0 + s // TR], idx_v.at[s % 2],
                isem.at[s % 2],
            )

        def idx_wait(s):
            pltpu.make_async_copy(
                idx_hbm.at[0, 0], idx_v.at[s % 2], isem.at[s % 2]
            ).wait()

        def gathers_start(s):
            p = s % 2
            for hf in range(2):
                for u in range(4):
                    pltpu.async_copy(
                        table_hbm.at[idx_v.at[p, hf * 4 + u]],
                        rows_v.at[p, hf, u],
                        gsem.at[p, hf],
                    )

        def gathers_wait(s, hf):
            p = s % 2
            for u in range(4):
                pltpu.make_async_copy(
                    table_hbm.at[idx_v.at[p, hf * 4 + u]],
                    rows_v.at[p, hf, u],
                    gsem.at[p, hf],
                ).wait()

        def write_start(s, hf):
            t0 = (s % TR) * 8 + hf * 4
            jb = jb0 + s // TR
            pltpu.async_copy(
                trans_v.at[hf],
                out_hbm.at[pl.ds(t0, 4), :, pl.ds(jb * 1024, 1024)],
                osem.at[hf],
            )

        def write_wait(hf):
            pltpu.make_async_copy(
                trans_v.at[hf],
                out_hbm.at[pl.ds(0, 4), :, pl.ds(0, 1024)],
                osem.at[hf],
            ).wait()

        pltpu.sync_copy(idx_hbm.at[0, jb0], idx_v.at[0])
        gathers_start(0)
        idx_start(1)

        def tblock(s, carry):
            @pl.when(s + 1 < NTB)
            def _():
                idx_wait(s + 1)
                gathers_start(s + 1)

            for hf in range(2):
                gathers_wait(s, hf)

                @pl.when(s >= 1)
                def _():
                    write_wait(hf)

                p = s % 2
                plsc.subcore_barrier()

                @functools.partial(plsc.parallel_loop, 0, 8, unroll=2)
                def _(b0):
                    bvec = iota + b0 * 16
                    for u in range(4):
                        for h in range(H):
                            hv = jnp.full((16,), h, jnp.int32)
                            x = plsc.load_gather(rows_v.at[p, hf, u], [bvec, hv])
                            trans_v[hf, u, h >> 3,
                                    pl.ds((h & 7) * 128 + b0 * 16, 16)] = x

                plsc.subcore_barrier()
                write_start(s, hf)

            @pl.when(s + 2 < NTB)
            def _():
                idx_start(s + 2)
            return carry

        lax.fori_loop(0, NTB, tblock, 0)
        for hf in range(2):
            write_wait(hf)

    return body(idx4, table)


def kernel(input_ids, W_embed):
    idx4 = input_ids.T.reshape(TR, 8, JB, 128).transpose(0, 2, 1, 3)
    out2 = _embed_lookup(idx4, W_embed)
    out = (
        out2.reshape(T, HB, JB, 8, 128)
        .transpose(2, 4, 0, 1, 3)
        .reshape(B, T, H)
    )
    return out
